# baseline (speedup 1.0000x reference)
import functools

import jax
import jax.numpy as jnp
from jax import lax
from jax.experimental import pallas as pl
from jax.experimental.pallas import tpu as pltpu
from jax.experimental.pallas import tpu_sc as plsc

_VOCAB = 100000
_EMBED = 64
_CONTEXT = 200
_HIDDEN = 128

_TILE = 4096
_NV = (_VOCAB + _TILE - 1) // _TILE
_PADDED = _NV * _TILE

_NC = 2
_NS = 16
_ROWS_PER_W = 8
_NW_ACTIVE = _CONTEXT // _ROWS_PER_W


def _sc_gather(emb_hbm, idx_hbm, out_hbm, idx_v, rows_v, sem):
    wid = lax.axis_index("s") * _NC + lax.axis_index("c")

    @pl.when(wid < _NW_ACTIVE)
    def _():
        base = wid * _ROWS_PER_W
        pltpu.sync_copy(idx_hbm.at[pl.ds(base, _ROWS_PER_W)], idx_v)
        pltpu.async_copy(emb_hbm.at[idx_v], rows_v, sem).wait()
        pltpu.sync_copy(rows_v, out_hbm.at[pl.ds(base, _ROWS_PER_W)])


@functools.cache
def _sc_gather_call():
    return pl.kernel(
        _sc_gather,
        out_type=jax.ShapeDtypeStruct((_CONTEXT, _EMBED), jnp.float32),
        mesh=plsc.VectorSubcoreMesh(core_axis_name="c", subcore_axis_name="s"),
        scratch_types=[
            pltpu.VMEM((_ROWS_PER_W,), jnp.int32),
            pltpu.VMEM((_ROWS_PER_W, _EMBED), jnp.float32),
            pltpu.SemaphoreType.DMA,
        ],
        compiler_params=pltpu.CompilerParams(use_tc_tiling_on_sc=False),
    )


def _tc_mlp(embeds_ref, w1_ref, b1_ref, w2_ref, b2_ref, out_ref,
            h_ref, logits_ref, m_ref, s_ref):
    p = pl.program_id(0)
    j = pl.program_id(1)

    @pl.when((p == 0) & (j == 0))
    def _():
        h = lax.dot_general(embeds_ref[...], w1_ref[...],
                            (((1,), (1,)), ((), ())),
                            preferred_element_type=jnp.float32) + b1_ref[...]
        h_ref[...] = jnp.maximum(h, 0.0)
        m_ref[0] = -jnp.inf
        s_ref[0] = 0.0

    @pl.when(p == 0)
    def _():
        t = lax.dot_general(h_ref[...], w2_ref[...],
                            (((1,), (1,)), ((), ())),
                            preferred_element_type=jnp.float32) + b2_ref[...]
        col = j * _TILE + lax.broadcasted_iota(jnp.int32, (1, _TILE), 1)
        t = jnp.where(col < _VOCAB, t, -jnp.inf)
        logits_ref[:, pl.ds(j * _TILE, _TILE)] = t
        tmax = jnp.max(t)
        m_new = jnp.maximum(m_ref[0], tmax)
        s_ref[0] = (s_ref[0] * jnp.exp(m_ref[0] - m_new)
                    + jnp.sum(jnp.exp(t - m_new)))
        m_ref[0] = m_new
        out_ref[...] = t

    @pl.when(p == 1)
    def _():
        lse = m_ref[0] + jnp.log(s_ref[0])
        out_ref[...] = logits_ref[:, pl.ds(j * _TILE, _TILE)] - lse


def _w2_index(p, j):
    return jnp.where(p == 0, j, _NV - 1)


@functools.cache
def _tc_mlp_call(interpret=False):
    return pl.pallas_call(
        _tc_mlp,
        grid=(2, _NV),
        in_specs=[
            pl.BlockSpec((1, _CONTEXT * _EMBED), lambda p, j: (0, 0)),
            pl.BlockSpec((_HIDDEN, _CONTEXT * _EMBED), lambda p, j: (0, 0)),
            pl.BlockSpec((1, _HIDDEN), lambda p, j: (0, 0)),
            pl.BlockSpec((_TILE, _HIDDEN), lambda p, j: (_w2_index(p, j), 0)),
            pl.BlockSpec((1, _TILE), lambda p, j: (0, _w2_index(p, j))),
        ],
        out_specs=pl.BlockSpec((1, _TILE), lambda p, j: (0, j)),
        out_shape=jax.ShapeDtypeStruct((1, _PADDED), jnp.float32),
        scratch_shapes=[
            pltpu.VMEM((1, _HIDDEN), jnp.float32),
            pltpu.VMEM((1, _PADDED), jnp.float32),
            pltpu.SMEM((1,), jnp.float32),
            pltpu.SMEM((1,), jnp.float32),
        ],
        compiler_params=pltpu.CompilerParams(
            dimension_semantics=("arbitrary", "arbitrary"),
        ),
        interpret=interpret,
    )


@jax.jit
def kernel(inputs, emb, W1, b1, W2, b2):
    embeds = _sc_gather_call()(emb, inputs)
    embeds = embeds.reshape(1, _CONTEXT * _EMBED)
    out = _tc_mlp_call()(embeds, W1, b1.reshape(1, _HIDDEN), W2,
                         b2.reshape(1, _VOCAB))
    return out[:, :_VOCAB]


# speedup vs baseline: 1.2659x; 1.2659x over previous
import functools

import jax
import jax.numpy as jnp
from jax import lax
from jax.experimental import pallas as pl
from jax.experimental.pallas import tpu as pltpu
from jax.experimental.pallas import tpu_sc as plsc

_VOCAB = 100000
_EMBED = 64
_CONTEXT = 200
_HIDDEN = 128

_TILE = 4096
_NV = (_VOCAB + _TILE - 1) // _TILE
_PADDED = _NV * _TILE

_NC = 2
_NS = 16
_ROWS_PER_W = 8
_NW_ACTIVE = _CONTEXT // _ROWS_PER_W


def _sc_gather(emb_hbm, idx_hbm, out_hbm, idx_v, rows_v, sem):
    wid = lax.axis_index("s") * _NC + lax.axis_index("c")

    @pl.when(wid < _NW_ACTIVE)
    def _():
        base = wid * _ROWS_PER_W
        pltpu.sync_copy(idx_hbm.at[pl.ds(base, 16)], idx_v)
        vec = idx_v[...]
        lane = lax.iota(jnp.int32, 16)
        copies = []
        for k in range(_ROWS_PER_W):
            row = jnp.sum(jnp.where(lane == k, vec, 0), axis=0)
            copies.append(pltpu.async_copy(
                emb_hbm.at[pl.ds(row, 1)], rows_v.at[pl.ds(k, 1)], sem))
        for c in copies:
            c.wait()
        pltpu.sync_copy(rows_v, out_hbm.at[pl.ds(base, _ROWS_PER_W)])


@functools.cache
def _sc_gather_call():
    return pl.kernel(
        _sc_gather,
        out_type=jax.ShapeDtypeStruct((_CONTEXT, _EMBED), jnp.float32),
        mesh=plsc.VectorSubcoreMesh(core_axis_name="c", subcore_axis_name="s"),
        scratch_types=[
            pltpu.VMEM((16,), jnp.int32),
            pltpu.VMEM((_ROWS_PER_W, _EMBED), jnp.float32),
            pltpu.SemaphoreType.DMA,
        ],
        compiler_params=pltpu.CompilerParams(needs_layout_passes=False),
    )


def _tc_mlp(embeds_ref, w1_ref, b1_ref, w2_ref, b2_ref, out_ref,
            h_ref, logits_ref, m_ref, s_ref):
    p = pl.program_id(0)
    j = pl.program_id(1)

    @pl.when((p == 0) & (j == 0))
    def _():
        h = lax.dot_general(embeds_ref[...], w1_ref[...],
                            (((1,), (1,)), ((), ())),
                            preferred_element_type=jnp.float32) + b1_ref[...]
        h_ref[...] = jnp.maximum(h, 0.0)
        m_ref[0] = -jnp.inf
        s_ref[0] = 0.0

    @pl.when(p == 0)
    def _():
        t = lax.dot_general(h_ref[...], w2_ref[...],
                            (((1,), (1,)), ((), ())),
                            preferred_element_type=jnp.float32) + b2_ref[...]
        col = j * _TILE + lax.broadcasted_iota(jnp.int32, (1, _TILE), 1)
        t = jnp.where(col < _VOCAB, t, -jnp.inf)
        logits_ref[:, pl.ds(j * _TILE, _TILE)] = t
        tmax = jnp.max(t)
        m_new = jnp.maximum(m_ref[0], tmax)
        s_ref[0] = (s_ref[0] * jnp.exp(m_ref[0] - m_new)
                    + jnp.sum(jnp.exp(t - m_new)))
        m_ref[0] = m_new
        out_ref[...] = t

    @pl.when(p == 1)
    def _():
        lse = m_ref[0] + jnp.log(s_ref[0])
        out_ref[...] = logits_ref[:, pl.ds(j * _TILE, _TILE)] - lse


def _w2_index(p, j):
    return jnp.where(p == 0, j, _NV - 1)


@functools.cache
def _tc_mlp_call(interpret=False):
    return pl.pallas_call(
        _tc_mlp,
        grid=(2, _NV),
        in_specs=[
            pl.BlockSpec((1, _CONTEXT * _EMBED), lambda p, j: (0, 0)),
            pl.BlockSpec((_HIDDEN, _CONTEXT * _EMBED), lambda p, j: (0, 0)),
            pl.BlockSpec((1, _HIDDEN), lambda p, j: (0, 0)),
            pl.BlockSpec((_TILE, _HIDDEN), lambda p, j: (_w2_index(p, j), 0)),
            pl.BlockSpec((1, _TILE), lambda p, j: (0, _w2_index(p, j))),
        ],
        out_specs=pl.BlockSpec((1, _TILE), lambda p, j: (0, j)),
        out_shape=jax.ShapeDtypeStruct((1, _PADDED), jnp.float32),
        scratch_shapes=[
            pltpu.VMEM((1, _HIDDEN), jnp.float32),
            pltpu.VMEM((1, _PADDED), jnp.float32),
            pltpu.SMEM((1,), jnp.float32),
            pltpu.SMEM((1,), jnp.float32),
        ],
        compiler_params=pltpu.CompilerParams(
            dimension_semantics=("arbitrary", "arbitrary"),
        ),
        interpret=interpret,
    )


@jax.jit
def kernel(inputs, emb, W1, b1, W2, b2):
    idx_pad = jnp.pad(inputs, (0, 16))
    embeds = _sc_gather_call()(emb, idx_pad)
    embeds = embeds.reshape(1, _CONTEXT * _EMBED)
    out = _tc_mlp_call()(embeds, W1, b1.reshape(1, _HIDDEN), W2,
                         b2.reshape(1, _VOCAB))
    return out[:, :_VOCAB]


# speedup vs baseline: 2.1711x; 1.7151x over previous
import functools

import jax
import jax.numpy as jnp
from jax import lax
from jax.experimental import pallas as pl
from jax.experimental.pallas import tpu as pltpu
from jax.experimental.pallas import tpu_sc as plsc

_VOCAB = 100000
_EMBED = 64
_CONTEXT = 200
_HIDDEN = 128
_FLAT = _CONTEXT * _EMBED

_TILE = 4096
_NV = (_VOCAB + _TILE - 1) // _TILE
_PADDED = _NV * _TILE

_NC = 2
_NS = 16
_ROWS_PER_W = 8
_NW_ACTIVE = _CONTEXT // _ROWS_PER_W


def _sc_gather(embt_hbm, idx_hbm, out_hbm, idx_v, land, flat_v, sem):
    wid = lax.axis_index("s") * _NC + lax.axis_index("c")

    @pl.when(wid < _NW_ACTIVE)
    def _():
        base = wid * _ROWS_PER_W
        pltpu.sync_copy(idx_hbm.at[pl.ds(base, _ROWS_PER_W)],
                        idx_v.at[pl.ds(0, _ROWS_PER_W)])
        vec = idx_v[...]
        lane = lax.iota(jnp.int32, 16)
        cols = []
        copies = []
        for k in range(_ROWS_PER_W):
            col = jnp.sum(jnp.where(lane == k, vec, 0), axis=0)
            col0 = pl.multiple_of((col // 128) * 128, 128)
            cols.append(col - col0)
            copies.append(pltpu.async_copy(
                embt_hbm.at[:, pl.ds(col0, 128)], land.at[k], sem))
        for c in copies:
            c.wait()
        for k in range(_ROWS_PER_W):
            lane_in_tile = jnp.full((16,), cols[k], dtype=jnp.int32)
            for c in range(_EMBED // 16):
                rows16 = lane + c * 16
                flat_v[pl.ds(k * _EMBED + c * 16, 16)] = plsc.load_gather(
                    land.at[k], [rows16, lane_in_tile])
        pltpu.sync_copy(flat_v, out_hbm.at[pl.ds(base * _EMBED,
                                                 _ROWS_PER_W * _EMBED)])


@functools.cache
def _sc_gather_call():
    return pl.kernel(
        _sc_gather,
        out_type=jax.ShapeDtypeStruct((_FLAT,), jnp.float32),
        mesh=plsc.VectorSubcoreMesh(core_axis_name="c", subcore_axis_name="s"),
        scratch_types=[
            pltpu.VMEM((16,), jnp.int32),
            pltpu.VMEM((_ROWS_PER_W, _EMBED, 128), jnp.float32),
            pltpu.VMEM((_ROWS_PER_W * _EMBED,), jnp.float32),
            pltpu.SemaphoreType.DMA,
        ],
        compiler_params=pltpu.CompilerParams(
            needs_layout_passes=False,
        ),
    )


def _tc_mlp(embeds_ref, w1_ref, b1_ref, w2_ref, b2_ref, out_ref,
            h_ref, logits_ref, m_ref, s_ref):
    j = pl.program_id(0)

    @pl.when(j == 0)
    def _():
        e = embeds_ref[...].reshape(1, _FLAT)
        h = lax.dot_general(e, w1_ref[...],
                            (((1,), (1,)), ((), ())),
                            preferred_element_type=jnp.float32) + b1_ref[...]
        h_ref[...] = jnp.maximum(h, 0.0)
        m_ref[0] = -jnp.inf
        s_ref[0] = 0.0

    t = (lax.dot_general(h_ref[...], w2_ref[...],
                         (((1,), (1,)), ((), ())),
                         preferred_element_type=jnp.float32)
         + b2_ref[...].reshape(1, _TILE))
    col = j * _TILE + lax.broadcasted_iota(jnp.int32, (1, _TILE), 1)
    t = jnp.where(col < _VOCAB, t, -jnp.inf)
    logits_ref[:, pl.ds(j * _TILE, _TILE)] = t
    tmax = jnp.max(t)
    m_new = jnp.maximum(m_ref[0], tmax)
    s_ref[0] = (s_ref[0] * jnp.exp(m_ref[0] - m_new)
                + jnp.sum(jnp.exp(t - m_new)))
    m_ref[0] = m_new

    @pl.when(j == _NV - 1)
    def _():
        lse = m_ref[0] + jnp.log(s_ref[0])
        out_ref[...] = logits_ref[:, pl.ds(0, _VOCAB)] - lse


@functools.cache
def _tc_mlp_call(interpret=False):
    return pl.pallas_call(
        _tc_mlp,
        grid=(_NV,),
        in_specs=[
            pl.BlockSpec((_FLAT,), lambda j: (0,)),
            pl.BlockSpec((_HIDDEN, _FLAT), lambda j: (0, 0)),
            pl.BlockSpec((1, _HIDDEN), lambda j: (0, 0)),
            pl.BlockSpec((_TILE, _HIDDEN), lambda j: (j, 0)),
            pl.BlockSpec((_TILE,), lambda j: (j,)),
        ],
        out_specs=pl.BlockSpec((1, _VOCAB), lambda j: (0, 0)),
        out_shape=jax.ShapeDtypeStruct((1, _VOCAB), jnp.float32),
        scratch_shapes=[
            pltpu.VMEM((1, _HIDDEN), jnp.float32),
            pltpu.VMEM((1, _PADDED), jnp.float32),
            pltpu.SMEM((1,), jnp.float32),
            pltpu.SMEM((1,), jnp.float32),
        ],
        compiler_params=pltpu.CompilerParams(
            dimension_semantics=("arbitrary",),
        ),
        interpret=interpret,
    )


@jax.jit
def kernel(inputs, emb, W1, b1, W2, b2):
    embeds = _sc_gather_call()(emb.T, inputs)
    return _tc_mlp_call()(embeds, W1, b1.reshape(1, _HIDDEN), W2, b2)
